# Initial kernel scaffold; baseline (speedup 1.0000x reference)
import jax
import jax.numpy as jnp
from jax.experimental import pallas as pl


def kernel(x, W1, g1, b1, W2, g2, b2, W3, g3, b3, W4, g4, b4):
    raise NotImplementedError("write your pallas kernel here")


# speedup vs baseline: 8.4539x; 8.4539x over previous
import functools

import jax
import jax.numpy as jnp
from jax.experimental import pallas as pl
from jax.experimental.pallas import tpu as pltpu

B = 8
N = 1024
K = 20
TR = 256
NEG = -1e30
BIGI = 2**30


def _normalize(u, mean_ref, var_ref, g_ref, bt_ref):
    t = (u - mean_ref[...]) / jnp.sqrt(var_ref[...] + 1e-5)
    t = t * g_ref[...] + bt_ref[...]
    return jnp.where(t > 0, t, 0.2 * t)


def _layer_body(first, cin, cout,
                u_full_ref, w_ref, mean_ref, var_ref, g_ref, bt_ref,
                u_out_ref, s1_ref, s2_ref,
                xn_s, sp_s):
    b = pl.program_id(0)
    r = pl.program_id(1)

    @pl.when(r == 0)
    def _():
        u = u_full_ref[0]
        if first:
            xn = u
        else:
            xn = _normalize(u, mean_ref, var_ref, g_ref, bt_ref)
        xn_s[...] = xn
        hi = xn.astype(jnp.bfloat16)
        rem = xn - hi.astype(jnp.float32)
        mid = rem.astype(jnp.bfloat16)
        lo = (rem - mid.astype(jnp.float32)).astype(jnp.bfloat16)
        sp_s[...] = jnp.concatenate([hi, mid, lo], axis=1)

    xn = xn_s[...]
    xn_t = xn_s[pl.ds(r * TR, TR), :]
    sp = sp_s[...]
    hi = sp[:, :cin]
    hi_t = sp_s[pl.ds(r * TR, TR), :cin]

    g = jax.lax.dot_general(
        hi_t, hi, (((1,), (1,)), ((), ())),
        preferred_element_type=jnp.float32)
    xx_row = jax.lax.dot_general(
        jnp.ones((1, cin), jnp.float32), xn * xn, (((1,), (1,)), ((), ())),
        preferred_element_type=jnp.float32,
        precision=jax.lax.Precision.HIGHEST)
    xx_t = jnp.sum(xn_t * xn_t, axis=1, keepdims=True)
    score = ((-xx_row) - (-2.0 * g)) - xx_t

    iota = jax.lax.broadcasted_iota(jnp.int32, (TR, N), 1)
    work = score
    m_acc = jnp.full((TR, cout), NEG, jnp.float32)
    s1_t = jnp.zeros((1, cout), jnp.float32)
    s2_t = jnp.zeros((1, cout), jnp.float32)
    for _ in range(K):
        m = jnp.max(work, axis=1, keepdims=True)
        eq = work == m
        idxj = jnp.min(jnp.where(eq, iota, BIGI), axis=1, keepdims=True)
        e = iota == idxj
        work = jnp.where(e, NEG, work)
        nb3 = jax.lax.dot_general(
            e.astype(jnp.bfloat16), sp, (((1,), (0,)), ((), ())),
            preferred_element_type=jnp.float32)
        nb = ((nb3[:, :cin] + nb3[:, cin:2 * cin])
              + nb3[:, 2 * cin:])
        diff = nb - xn_t
        feat = jnp.concatenate([diff.astype(jnp.bfloat16), hi_t], axis=1)
        yj = jax.lax.dot_general(
            feat, w_ref[...], (((1,), (1,)), ((), ())),
            preferred_element_type=jnp.float32)
        m_acc = jnp.maximum(m_acc, yj)
        s1_t = s1_t + jnp.sum(yj, axis=0, keepdims=True)
        s2_t = s2_t + jnp.sum(yj * yj, axis=0, keepdims=True)

    @pl.when(jnp.logical_and(b == 0, r == 0))
    def _():
        s1_ref[...] = jnp.zeros_like(s1_ref)
        s2_ref[...] = jnp.zeros_like(s2_ref)

    s1_ref[...] += s1_t
    s2_ref[...] += s2_t
    u_out_ref[0] = m_acc


def _layer_call(u, w, mean, var, g, bt, first):
    cin = u.shape[-1]
    cout = w.shape[0]
    body = functools.partial(_layer_body, first, cin, cout)
    grid = (B, N // TR)
    return pl.pallas_call(
        body,
        grid=grid,
        in_specs=[
            pl.BlockSpec((1, N, cin), lambda b, r: (b, 0, 0)),
            pl.BlockSpec((cout, 2 * cin), lambda b, r: (0, 0)),
            pl.BlockSpec((1, cin), lambda b, r: (0, 0)),
            pl.BlockSpec((1, cin), lambda b, r: (0, 0)),
            pl.BlockSpec((1, cin), lambda b, r: (0, 0)),
            pl.BlockSpec((1, cin), lambda b, r: (0, 0)),
        ],
        out_specs=[
            pl.BlockSpec((1, TR, cout), lambda b, r: (b, r, 0)),
            pl.BlockSpec((1, cout), lambda b, r: (0, 0)),
            pl.BlockSpec((1, cout), lambda b, r: (0, 0)),
        ],
        out_shape=[
            jax.ShapeDtypeStruct((B, N, cout), jnp.float32),
            jax.ShapeDtypeStruct((1, cout), jnp.float32),
            jax.ShapeDtypeStruct((1, cout), jnp.float32),
        ],
        scratch_shapes=[
            pltpu.VMEM((N, cin), jnp.float32),
            pltpu.VMEM((N, 3 * cin), jnp.bfloat16),
        ],
    )(u, w, mean, var, g, bt)


def _epilogue_body(u1_ref, u2_ref, u3_ref, u4_ref,
                   m1_ref, v1_ref, g1_ref, b1_ref,
                   m2_ref, v2_ref, g2_ref, b2_ref,
                   m3_ref, v3_ref, g3_ref, b3_ref,
                   m4_ref, v4_ref, g4_ref, b4_ref, out_ref):
    outs = []
    for u_ref, m_ref, v_ref, g_ref, bt_ref in (
            (u1_ref, m1_ref, v1_ref, g1_ref, b1_ref),
            (u2_ref, m2_ref, v2_ref, g2_ref, b2_ref),
            (u3_ref, m3_ref, v3_ref, g3_ref, b3_ref),
            (u4_ref, m4_ref, v4_ref, g4_ref, b4_ref)):
        y = _normalize(u_ref[0], m_ref, v_ref, g_ref, bt_ref)
        outs.append(jnp.max(y, axis=0, keepdims=True))
    out_ref[0] = jnp.concatenate(outs, axis=1)


def _epilogue(us, stats):
    args = [us[0], us[1], us[2], us[3]]
    specs = [pl.BlockSpec((1, N, u.shape[-1]), lambda b: (b, 0, 0))
             for u in us]
    for st in stats:
        for a in st:
            args.append(a)
            c = a.shape[-1]
            specs.append(pl.BlockSpec((1, c), lambda b: (0, 0)))
    return pl.pallas_call(
        _epilogue_body,
        grid=(B,),
        in_specs=specs,
        out_specs=pl.BlockSpec((1, 1, 512), lambda b: (b, 0, 0)),
        out_shape=jax.ShapeDtypeStruct((B, 1, 512), jnp.float32),
    )(*args)


def _moments(s1, s2):
    n = jnp.float32(B * N * K)
    mean = s1 / n
    var = s2 / n - mean * mean
    return mean, var


def kernel(x, W1, g1, b1, W2, g2, b2, W3, g3, b3, W4, g4, b4):
    x8 = jnp.pad(x, ((0, 0), (0, 0), (0, 5)))
    w1p = jnp.zeros((64, 16), jnp.float32)
    w1p = w1p.at[:, 0:3].set(W1[:, 0:3]).at[:, 8:11].set(W1[:, 3:6])

    dummy = jnp.zeros((1, 8), jnp.float32)
    bf = jnp.bfloat16

    u1, s11, s21 = _layer_call(x8, w1p.astype(bf), dummy, dummy, dummy,
                               dummy, True)
    m1, v1 = _moments(s11, s21)
    st1 = (m1, v1, g1[None, :], b1[None, :])
    u2, s12, s22 = _layer_call(u1, W2.astype(bf), *st1, False)
    m2, v2 = _moments(s12, s22)
    st2 = (m2, v2, g2[None, :], b2[None, :])
    u3, s13, s23 = _layer_call(u2, W3.astype(bf), *st2, False)
    m3, v3 = _moments(s13, s23)
    st3 = (m3, v3, g3[None, :], b3[None, :])
    u4, s14, s24 = _layer_call(u3, W4.astype(bf), *st3, False)
    m4, v4 = _moments(s14, s24)
    st4 = (m4, v4, g4[None, :], b4[None, :])

    return _epilogue((u1, u2, u3, u4), (st1, st2, st3, st4))


# speedup vs baseline: 8.9149x; 1.0545x over previous
import functools

import jax
import jax.numpy as jnp
from jax.experimental import pallas as pl
from jax.experimental.pallas import tpu as pltpu

B = 8
N = 1024
K = 20
TR = 512
NEG = -1e30
BIGI = 2**30


def _normalize(u, mean_ref, var_ref, g_ref, bt_ref):
    t = (u - mean_ref[...]) / jnp.sqrt(var_ref[...] + 1e-5)
    t = t * g_ref[...] + bt_ref[...]
    return jnp.where(t > 0, t, 0.2 * t)


def _layer_body(first, cin, cout,
                u_full_ref, w_ref, mean_ref, var_ref, g_ref, bt_ref,
                u_out_ref, s1_ref, s2_ref,
                xn_s, sp_s):
    b = pl.program_id(0)
    r = pl.program_id(1)

    @pl.when(r == 0)
    def _():
        u = u_full_ref[0]
        if first:
            xn = u
        else:
            xn = _normalize(u, mean_ref, var_ref, g_ref, bt_ref)
        xn_s[...] = xn
        hi = xn.astype(jnp.bfloat16)
        rem = xn - hi.astype(jnp.float32)
        mid = rem.astype(jnp.bfloat16)
        lo = (rem - mid.astype(jnp.float32)).astype(jnp.bfloat16)
        sp_s[...] = jnp.concatenate([hi, mid, lo], axis=1)

    xn = xn_s[...]
    xn_t = xn_s[pl.ds(r * TR, TR), :]
    sp = sp_s[...]
    hi = sp[:, :cin]
    hi_t = sp_s[pl.ds(r * TR, TR), :cin]

    g = jax.lax.dot_general(
        hi_t, hi, (((1,), (1,)), ((), ())),
        preferred_element_type=jnp.float32)
    xx_row = jax.lax.dot_general(
        jnp.ones((1, cin), jnp.float32), xn * xn, (((1,), (1,)), ((), ())),
        preferred_element_type=jnp.float32,
        precision=jax.lax.Precision.HIGHEST)
    xx_t = jnp.sum(xn_t * xn_t, axis=1, keepdims=True)
    score = ((-xx_row) - (-2.0 * g)) - xx_t

    iota = jax.lax.broadcasted_iota(jnp.int32, (TR, N), 1)
    work = score
    m_acc = jnp.full((TR, cout), NEG, jnp.float32)
    s1_t = jnp.zeros((1, cout), jnp.float32)
    s2_t = jnp.zeros((1, cout), jnp.float32)
    for _ in range(K):
        m = jnp.max(work, axis=1, keepdims=True)
        eq = work == m
        idxj = jnp.min(jnp.where(eq, iota, BIGI), axis=1, keepdims=True)
        e = iota == idxj
        work = jnp.where(e, NEG, work)
        nb3 = jax.lax.dot_general(
            e.astype(jnp.bfloat16), sp, (((1,), (0,)), ((), ())),
            preferred_element_type=jnp.float32)
        nb = ((nb3[:, :cin] + nb3[:, cin:2 * cin])
              + nb3[:, 2 * cin:])
        diff = nb - xn_t
        feat = jnp.concatenate([diff.astype(jnp.bfloat16), hi_t], axis=1)
        yj = jax.lax.dot_general(
            feat, w_ref[...], (((1,), (1,)), ((), ())),
            preferred_element_type=jnp.float32)
        m_acc = jnp.maximum(m_acc, yj)
        s1_t = s1_t + jnp.sum(yj, axis=0, keepdims=True)
        s2_t = s2_t + jnp.sum(yj * yj, axis=0, keepdims=True)

    @pl.when(jnp.logical_and(b == 0, r == 0))
    def _():
        s1_ref[...] = jnp.zeros_like(s1_ref)
        s2_ref[...] = jnp.zeros_like(s2_ref)

    s1_ref[...] += s1_t
    s2_ref[...] += s2_t
    u_out_ref[0] = m_acc


def _layer_call(u, w, mean, var, g, bt, first):
    cin = u.shape[-1]
    cout = w.shape[0]
    body = functools.partial(_layer_body, first, cin, cout)
    grid = (B, N // TR)
    return pl.pallas_call(
        body,
        grid=grid,
        in_specs=[
            pl.BlockSpec((1, N, cin), lambda b, r: (b, 0, 0)),
            pl.BlockSpec((cout, 2 * cin), lambda b, r: (0, 0)),
            pl.BlockSpec((1, cin), lambda b, r: (0, 0)),
            pl.BlockSpec((1, cin), lambda b, r: (0, 0)),
            pl.BlockSpec((1, cin), lambda b, r: (0, 0)),
            pl.BlockSpec((1, cin), lambda b, r: (0, 0)),
        ],
        out_specs=[
            pl.BlockSpec((1, TR, cout), lambda b, r: (b, r, 0)),
            pl.BlockSpec((1, cout), lambda b, r: (0, 0)),
            pl.BlockSpec((1, cout), lambda b, r: (0, 0)),
        ],
        out_shape=[
            jax.ShapeDtypeStruct((B, N, cout), jnp.float32),
            jax.ShapeDtypeStruct((1, cout), jnp.float32),
            jax.ShapeDtypeStruct((1, cout), jnp.float32),
        ],
        scratch_shapes=[
            pltpu.VMEM((N, cin), jnp.float32),
            pltpu.VMEM((N, 3 * cin), jnp.bfloat16),
        ],
    )(u, w, mean, var, g, bt)


def _epilogue_body(u1_ref, u2_ref, u3_ref, u4_ref,
                   m1_ref, v1_ref, g1_ref, b1_ref,
                   m2_ref, v2_ref, g2_ref, b2_ref,
                   m3_ref, v3_ref, g3_ref, b3_ref,
                   m4_ref, v4_ref, g4_ref, b4_ref, out_ref):
    outs = []
    for u_ref, m_ref, v_ref, g_ref, bt_ref in (
            (u1_ref, m1_ref, v1_ref, g1_ref, b1_ref),
            (u2_ref, m2_ref, v2_ref, g2_ref, b2_ref),
            (u3_ref, m3_ref, v3_ref, g3_ref, b3_ref),
            (u4_ref, m4_ref, v4_ref, g4_ref, b4_ref)):
        y = _normalize(u_ref[0], m_ref, v_ref, g_ref, bt_ref)
        outs.append(jnp.max(y, axis=0, keepdims=True))
    out_ref[0] = jnp.concatenate(outs, axis=1)


def _epilogue(us, stats):
    args = [us[0], us[1], us[2], us[3]]
    specs = [pl.BlockSpec((1, N, u.shape[-1]), lambda b: (b, 0, 0))
             for u in us]
    for st in stats:
        for a in st:
            args.append(a)
            c = a.shape[-1]
            specs.append(pl.BlockSpec((1, c), lambda b: (0, 0)))
    return pl.pallas_call(
        _epilogue_body,
        grid=(B,),
        in_specs=specs,
        out_specs=pl.BlockSpec((1, 1, 512), lambda b: (b, 0, 0)),
        out_shape=jax.ShapeDtypeStruct((B, 1, 512), jnp.float32),
    )(*args)


def _moments(s1, s2):
    n = jnp.float32(B * N * K)
    mean = s1 / n
    var = s2 / n - mean * mean
    return mean, var


def kernel(x, W1, g1, b1, W2, g2, b2, W3, g3, b3, W4, g4, b4):
    x8 = jnp.pad(x, ((0, 0), (0, 0), (0, 5)))
    w1p = jnp.zeros((64, 16), jnp.float32)
    w1p = w1p.at[:, 0:3].set(W1[:, 0:3]).at[:, 8:11].set(W1[:, 3:6])

    dummy = jnp.zeros((1, 8), jnp.float32)
    bf = jnp.bfloat16

    u1, s11, s21 = _layer_call(x8, w1p.astype(bf), dummy, dummy, dummy,
                               dummy, True)
    m1, v1 = _moments(s11, s21)
    st1 = (m1, v1, g1[None, :], b1[None, :])
    u2, s12, s22 = _layer_call(u1, W2.astype(bf), *st1, False)
    m2, v2 = _moments(s12, s22)
    st2 = (m2, v2, g2[None, :], b2[None, :])
    u3, s13, s23 = _layer_call(u2, W3.astype(bf), *st2, False)
    m3, v3 = _moments(s13, s23)
    st3 = (m3, v3, g3[None, :], b3[None, :])
    u4, s14, s24 = _layer_call(u3, W4.astype(bf), *st3, False)
    m4, v4 = _moments(s14, s24)
    st4 = (m4, v4, g4[None, :], b4[None, :])

    return _epilogue((u1, u2, u3, u4), (st1, st2, st3, st4))


# speedup vs baseline: 10.2659x; 1.1515x over previous
import functools

import jax
import jax.numpy as jnp
from jax.experimental import pallas as pl
from jax.experimental.pallas import tpu as pltpu

B = 8
N = 1024
K = 20
TR = 512
NEG = -1e30
BIGF = float(2**30)


def _normalize(u, mean_ref, var_ref, g_ref, bt_ref):
    t = (u - mean_ref[...]) / jnp.sqrt(var_ref[...] + 1e-5)
    t = t * g_ref[...] + bt_ref[...]
    return jnp.where(t > 0, t, 0.2 * t)


def _layer_body(first, cin, cout,
                u_full_ref, w_ref, mean_ref, var_ref, g_ref, bt_ref,
                u_out_ref, s1_ref, s2_ref,
                xn_s, sp_s):
    b = pl.program_id(0)
    r = pl.program_id(1)

    @pl.when(r == 0)
    def _():
        u = u_full_ref[0]
        if first:
            xn = u
        else:
            xn = _normalize(u, mean_ref, var_ref, g_ref, bt_ref)
        xn_s[...] = xn
        hi = xn.astype(jnp.bfloat16)
        rem = xn - hi.astype(jnp.float32)
        mid = rem.astype(jnp.bfloat16)
        lo = (rem - mid.astype(jnp.float32)).astype(jnp.bfloat16)
        sp_s[...] = jnp.concatenate([hi, mid, lo], axis=1)

    xn = xn_s[...]
    xn_t = xn_s[pl.ds(r * TR, TR), :]
    sp = sp_s[...]
    hi = sp[:, :cin]
    hi_t = sp_s[pl.ds(r * TR, TR), :cin]

    g = jax.lax.dot_general(
        hi_t, hi, (((1,), (1,)), ((), ())),
        preferred_element_type=jnp.float32)
    xx_row = jax.lax.dot_general(
        jnp.ones((1, cin), jnp.float32), xn * xn, (((1,), (1,)), ((), ())),
        preferred_element_type=jnp.float32,
        precision=jax.lax.Precision.HIGHEST)
    xx_t = jnp.sum(xn_t * xn_t, axis=1, keepdims=True)
    score = ((-xx_row) - (-2.0 * g)) - xx_t

    iota = jax.lax.broadcasted_iota(jnp.int32, (TR, N), 1).astype(jnp.float32)
    work = score
    m_acc = jnp.full((TR, cout), NEG, jnp.float32)
    s1_t = jnp.zeros((1, cout), jnp.float32)
    s2_t = jnp.zeros((1, cout), jnp.float32)
    for _ in range(K):
        m = jnp.max(work, axis=1, keepdims=True)
        eq = work == m
        idxj = jnp.min(jnp.where(eq, iota, BIGF), axis=1, keepdims=True)
        e = iota == idxj
        work = jnp.where(e, NEG, work)
        nb3 = jax.lax.dot_general(
            e.astype(jnp.bfloat16), sp, (((1,), (0,)), ((), ())),
            preferred_element_type=jnp.float32)
        nb = ((nb3[:, :cin] + nb3[:, cin:2 * cin])
              + nb3[:, 2 * cin:])
        diff = nb - xn_t
        feat = jnp.concatenate([diff.astype(jnp.bfloat16), hi_t], axis=1)
        yj = jax.lax.dot_general(
            feat, w_ref[...], (((1,), (1,)), ((), ())),
            preferred_element_type=jnp.float32)
        m_acc = jnp.maximum(m_acc, yj)
        s1_t = s1_t + jnp.sum(yj, axis=0, keepdims=True)
        s2_t = s2_t + jnp.sum(yj * yj, axis=0, keepdims=True)

    @pl.when(jnp.logical_and(b == 0, r == 0))
    def _():
        s1_ref[...] = jnp.zeros_like(s1_ref)
        s2_ref[...] = jnp.zeros_like(s2_ref)

    s1_ref[...] += s1_t
    s2_ref[...] += s2_t
    u_out_ref[0] = m_acc


def _layer_call(u, w, mean, var, g, bt, first):
    cin = u.shape[-1]
    cout = w.shape[0]
    body = functools.partial(_layer_body, first, cin, cout)
    grid = (B, N // TR)
    return pl.pallas_call(
        body,
        grid=grid,
        in_specs=[
            pl.BlockSpec((1, N, cin), lambda b, r: (b, 0, 0)),
            pl.BlockSpec((cout, 2 * cin), lambda b, r: (0, 0)),
            pl.BlockSpec((1, cin), lambda b, r: (0, 0)),
            pl.BlockSpec((1, cin), lambda b, r: (0, 0)),
            pl.BlockSpec((1, cin), lambda b, r: (0, 0)),
            pl.BlockSpec((1, cin), lambda b, r: (0, 0)),
        ],
        out_specs=[
            pl.BlockSpec((1, TR, cout), lambda b, r: (b, r, 0)),
            pl.BlockSpec((1, cout), lambda b, r: (0, 0)),
            pl.BlockSpec((1, cout), lambda b, r: (0, 0)),
        ],
        out_shape=[
            jax.ShapeDtypeStruct((B, N, cout), jnp.float32),
            jax.ShapeDtypeStruct((1, cout), jnp.float32),
            jax.ShapeDtypeStruct((1, cout), jnp.float32),
        ],
        scratch_shapes=[
            pltpu.VMEM((N, cin), jnp.float32),
            pltpu.VMEM((N, 3 * cin), jnp.bfloat16),
        ],
    )(u, w, mean, var, g, bt)


def _epilogue_body(u1_ref, u2_ref, u3_ref, u4_ref,
                   m1_ref, v1_ref, g1_ref, b1_ref,
                   m2_ref, v2_ref, g2_ref, b2_ref,
                   m3_ref, v3_ref, g3_ref, b3_ref,
                   m4_ref, v4_ref, g4_ref, b4_ref, out_ref):
    outs = []
    for u_ref, m_ref, v_ref, g_ref, bt_ref in (
            (u1_ref, m1_ref, v1_ref, g1_ref, b1_ref),
            (u2_ref, m2_ref, v2_ref, g2_ref, b2_ref),
            (u3_ref, m3_ref, v3_ref, g3_ref, b3_ref),
            (u4_ref, m4_ref, v4_ref, g4_ref, b4_ref)):
        y = _normalize(u_ref[0], m_ref, v_ref, g_ref, bt_ref)
        outs.append(jnp.max(y, axis=0, keepdims=True))
    out_ref[0] = jnp.concatenate(outs, axis=1)


def _epilogue(us, stats):
    args = [us[0], us[1], us[2], us[3]]
    specs = [pl.BlockSpec((1, N, u.shape[-1]), lambda b: (b, 0, 0))
             for u in us]
    for st in stats:
        for a in st:
            args.append(a)
            c = a.shape[-1]
            specs.append(pl.BlockSpec((1, c), lambda b: (0, 0)))
    return pl.pallas_call(
        _epilogue_body,
        grid=(B,),
        in_specs=specs,
        out_specs=pl.BlockSpec((1, 1, 512), lambda b: (b, 0, 0)),
        out_shape=jax.ShapeDtypeStruct((B, 1, 512), jnp.float32),
    )(*args)


def _moments(s1, s2):
    n = jnp.float32(B * N * K)
    mean = s1 / n
    var = s2 / n - mean * mean
    return mean, var


def kernel(x, W1, g1, b1, W2, g2, b2, W3, g3, b3, W4, g4, b4):
    x8 = jnp.pad(x, ((0, 0), (0, 0), (0, 5)))
    w1p = jnp.zeros((64, 16), jnp.float32)
    w1p = w1p.at[:, 0:3].set(W1[:, 0:3]).at[:, 8:11].set(W1[:, 3:6])

    dummy = jnp.zeros((1, 8), jnp.float32)
    bf = jnp.bfloat16

    u1, s11, s21 = _layer_call(x8, w1p.astype(bf), dummy, dummy, dummy,
                               dummy, True)
    m1, v1 = _moments(s11, s21)
    st1 = (m1, v1, g1[None, :], b1[None, :])
    u2, s12, s22 = _layer_call(u1, W2.astype(bf), *st1, False)
    m2, v2 = _moments(s12, s22)
    st2 = (m2, v2, g2[None, :], b2[None, :])
    u3, s13, s23 = _layer_call(u2, W3.astype(bf), *st2, False)
    m3, v3 = _moments(s13, s23)
    st3 = (m3, v3, g3[None, :], b3[None, :])
    u4, s14, s24 = _layer_call(u3, W4.astype(bf), *st3, False)
    m4, v4 = _moments(s14, s24)
    st4 = (m4, v4, g4[None, :], b4[None, :])

    return _epilogue((u1, u2, u3, u4), (st1, st2, st3, st4))
